# Initial kernel scaffold; baseline (speedup 1.0000x reference)
import jax
import jax.numpy as jnp
from jax.experimental import pallas as pl


def kernel(x, batch, W_ih, W_hh, b_ih, b_hh):
    raise NotImplementedError("write your pallas kernel here")


# speedup vs baseline: 24.4902x; 24.4902x over previous
import functools

import jax
import jax.numpy as jnp
from jax import lax
from jax.experimental import pallas as pl
from jax.experimental.pallas import tpu as pltpu

_N = 50000
_F = 512
_G = 256
_STEPS = 6
_BLK = 2000
_NBLK = _N // _BLK
_NEG = -1e30


def _step_kernel(x_ref, b_ref, wih_ref, whh_ref, bih_ref, bhh_ref,
                 qp_ref, hp_ref, cp_ref,
                 qs_out, h_out, c_out,
                 h_s, m_s, d_s, r_s):
    i = pl.program_id(0)

    @pl.when(i == 0)
    def _lstm():
        gates = (
            lax.dot_general(qp_ref[...], wih_ref[...],
                            (((1,), (1,)), ((), ())),
                            preferred_element_type=jnp.float32)
            + lax.dot_general(hp_ref[...], whh_ref[...],
                              (((1,), (1,)), ((), ())),
                              preferred_element_type=jnp.float32)
            + bih_ref[...] + bhh_ref[...]
        )
        i_g = gates[:, :_F]
        f_g = gates[:, _F:2 * _F]
        g_g = gates[:, 2 * _F:3 * _F]
        o_g = gates[:, 3 * _F:]
        c_new = jax.nn.sigmoid(f_g) * cp_ref[...] + jax.nn.sigmoid(i_g) * jnp.tanh(g_g)
        h_new = jax.nn.sigmoid(o_g) * jnp.tanh(c_new)
        h_s[...] = h_new
        h_out[...] = h_new
        c_out[...] = c_new
        m_s[...] = jnp.full((_G, 1), _NEG, jnp.float32)
        d_s[...] = jnp.zeros((_G, 1), jnp.float32)
        r_s[...] = jnp.zeros((_G, _F), jnp.float32)

    h = h_s[...]
    xb = x_ref[...]
    ids = b_ref[0]
    et = lax.dot_general(h, xb, (((1,), (1,)), ((), ())),
                         preferred_element_type=jnp.float32)
    gi = lax.broadcasted_iota(jnp.int32, (_G, _BLK), 0)
    og = gi == ids
    m_blk = jnp.max(jnp.where(og, et, _NEG), axis=1, keepdims=True)
    m_old = m_s[...]
    m_new = jnp.maximum(m_old, m_blk)
    alpha = jnp.exp(m_old - m_new)
    p = jnp.where(og, jnp.exp(et - m_new), 0.0)
    d_blk = jnp.sum(p, axis=1, keepdims=True)
    r_s[...] = r_s[...] * alpha + lax.dot_general(
        p, xb, (((1,), (0,)), ((), ())), preferred_element_type=jnp.float32)
    d_s[...] = d_s[...] * alpha + d_blk
    m_s[...] = m_new

    @pl.when(i == _NBLK - 1)
    def _finish():
        r = r_s[...] / (d_s[...] + 1e-16)
        qs_out[...] = jnp.concatenate([h_s[...], r], axis=1)


@functools.partial(jax.jit, static_argnames=())
def _run(x, batch3, w_ih, w_hh, b_ih2, b_hh2):
    q_star = jnp.zeros((_G, 2 * _F), jnp.float32)
    h = jnp.zeros((_G, _F), jnp.float32)
    c = jnp.zeros((_G, _F), jnp.float32)
    call = pl.pallas_call(
        _step_kernel,
        grid=(_NBLK,),
        in_specs=[
            pl.BlockSpec((_BLK, _F), lambda i: (i, 0)),
            pl.BlockSpec((1, 1, _BLK), lambda i: (i, 0, 0)),
            pl.BlockSpec((4 * _F, 2 * _F), lambda i: (0, 0)),
            pl.BlockSpec((4 * _F, _F), lambda i: (0, 0)),
            pl.BlockSpec((1, 4 * _F), lambda i: (0, 0)),
            pl.BlockSpec((1, 4 * _F), lambda i: (0, 0)),
            pl.BlockSpec((_G, 2 * _F), lambda i: (0, 0)),
            pl.BlockSpec((_G, _F), lambda i: (0, 0)),
            pl.BlockSpec((_G, _F), lambda i: (0, 0)),
        ],
        out_specs=[
            pl.BlockSpec((_G, 2 * _F), lambda i: (0, 0)),
            pl.BlockSpec((_G, _F), lambda i: (0, 0)),
            pl.BlockSpec((_G, _F), lambda i: (0, 0)),
        ],
        out_shape=[
            jax.ShapeDtypeStruct((_G, 2 * _F), jnp.float32),
            jax.ShapeDtypeStruct((_G, _F), jnp.float32),
            jax.ShapeDtypeStruct((_G, _F), jnp.float32),
        ],
        scratch_shapes=[
            pltpu.VMEM((_G, _F), jnp.float32),
            pltpu.VMEM((_G, 1), jnp.float32),
            pltpu.VMEM((_G, 1), jnp.float32),
            pltpu.VMEM((_G, _F), jnp.float32),
        ],
    )
    for _ in range(_STEPS):
        q_star, h, c = call(x, batch3, w_ih, w_hh, b_ih2, b_hh2, q_star, h, c)
    return q_star


def kernel(x, batch, W_ih, W_hh, b_ih, b_hh):
    batch3 = batch.astype(jnp.int32).reshape(_NBLK, 1, _BLK)
    return _run(x, batch3, W_ih, W_hh,
                b_ih.reshape(1, -1), b_hh.reshape(1, -1))
